# Initial kernel scaffold; baseline (speedup 1.0000x reference)
import jax
import jax.numpy as jnp
from jax.experimental import pallas as pl


def kernel(x, edge_index, batch, W0, b0, g0, be0, W1, b1, g1, be1, W2, b2, g2, be2, W3, b3, g3, be3, Wout, bout):
    raise NotImplementedError("write your pallas kernel here")


# speedup vs baseline: 1.5370x; 1.5370x over previous
import functools

import jax
import jax.numpy as jnp
from jax.experimental import pallas as pl
from jax.experimental.pallas import tpu as pltpu

N = 10000
G = 64
ROWS = 1000


def _mm_kernel(x_ref, w_ref, o_ref):
    o_ref[...] = jax.lax.dot_general(
        x_ref[...], w_ref[...], (((1,), (1,)), ((), ())),
        preferred_element_type=jnp.float32,
        precision=jax.lax.Precision.DEFAULT,
    )


def _matmul(x, W):
    n, d_in = x.shape
    d_out = W.shape[0]
    return pl.pallas_call(
        _mm_kernel,
        grid=(n // ROWS,),
        in_specs=[
            pl.BlockSpec((ROWS, d_in), lambda i: (i, 0)),
            pl.BlockSpec((d_out, d_in), lambda i: (0, 0)),
        ],
        out_specs=pl.BlockSpec((ROWS, d_out), lambda i: (i, 0)),
        out_shape=jax.ShapeDtypeStruct((n, d_out), jnp.float32),
    )(x, W)


def _stats_kernel(x_ref, o_ref):
    i = pl.program_id(0)

    @pl.when(i == 0)
    def _():
        o_ref[...] = jnp.zeros_like(o_ref)

    o_ref[0:1, :] += jnp.sum(x_ref[...], axis=0, keepdims=True)
    o_ref[1:2, :] += jnp.sum(x_ref[...] ** 2, axis=0, keepdims=True)


def _stats(x):
    n, d = x.shape
    return pl.pallas_call(
        _stats_kernel,
        grid=(n // ROWS,),
        in_specs=[pl.BlockSpec((ROWS, d), lambda i: (i, 0))],
        out_specs=pl.BlockSpec((8, d), lambda i: (0, 0)),
        out_shape=jax.ShapeDtypeStruct((8, d), jnp.float32),
    )(x)


def _bngelu_kernel(x_ref, st_ref, g_ref, be_ref, o_ref, *, n):
    mean = st_ref[0:1, :] / n
    var = st_ref[1:2, :] / n - mean * mean
    inv = jax.lax.rsqrt(var + 1e-5)
    y = (x_ref[...] - mean) * (inv * g_ref[...]) + be_ref[...]
    o_ref[...] = jax.nn.gelu(y)


def _bngelu(x, st, g, be):
    n, d = x.shape
    return pl.pallas_call(
        functools.partial(_bngelu_kernel, n=n),
        grid=(n // ROWS,),
        in_specs=[
            pl.BlockSpec((ROWS, d), lambda i: (i, 0)),
            pl.BlockSpec((8, d), lambda i: (0, 0)),
            pl.BlockSpec((1, d), lambda i: (0, 0)),
            pl.BlockSpec((1, d), lambda i: (0, 0)),
        ],
        out_specs=pl.BlockSpec((ROWS, d), lambda i: (i, 0)),
        out_shape=jax.ShapeDtypeStruct((n, d), jnp.float32),
    )(x, st, g.reshape(1, d), be.reshape(1, d))


def _pool_kernel(h_ref, w_ref, b_ref, bout_ref, o_ref):
    i = pl.program_id(0)

    @pl.when(i == 0)
    def _():
        o_ref[...] = jnp.broadcast_to(bout_ref[...], o_ref.shape)

    s = jax.lax.dot_general(
        h_ref[...], w_ref[...], (((1,), (1,)), ((), ())),
        preferred_element_type=jnp.float32,
        precision=jax.lax.Precision.DEFAULT,
    )
    seg = jax.lax.broadcasted_iota(jnp.int32, (h_ref.shape[0], G), 1)
    onehot = (b_ref[0, 0, :].reshape(-1, 1) == seg).astype(jnp.float32)
    o_ref[...] += jax.lax.dot_general(
        s, onehot, (((0,), (0,)), ((), ())),
        preferred_element_type=jnp.float32,
        precision=jax.lax.Precision.DEFAULT,
    )


def _pool_head(h, batch, Wout, bout):
    n, d = h.shape
    out = pl.pallas_call(
        _pool_kernel,
        grid=(n // ROWS,),
        in_specs=[
            pl.BlockSpec((ROWS, d), lambda i: (i, 0)),
            pl.BlockSpec((1, d), lambda i: (0, 0)),
            pl.BlockSpec((1, 1, ROWS), lambda i: (i, 0, 0)),
            pl.BlockSpec((1, 1), lambda i: (0, 0)),
        ],
        out_specs=pl.BlockSpec((1, G), lambda i: (0, 0)),
        out_shape=jax.ShapeDtypeStruct((1, G), jnp.float32),
    )(h, Wout, batch.reshape(n // ROWS, 1, ROWS), bout.reshape(1, 1))
    return out.reshape(G)


def _agg(h, src, dst, dis):
    msg = h[src] * (dis[src] * dis[dst])[:, None]
    out = jnp.zeros_like(h).at[dst].add(msg)
    return out + h * (dis * dis)[:, None]


def kernel(x, edge_index, batch, W0, b0, g0, be0, W1, b1, g1, be1,
           W2, b2, g2, be2, W3, b3, g3, be3, Wout, bout):
    src = edge_index[0]
    dst = edge_index[1]
    deg = jnp.ones((N,), jnp.float32).at[dst].add(1.0)
    dis = jax.lax.rsqrt(deg)

    h = x
    for W, g, be in ((W0, g0, be0), (W1, g1, be1), (W2, g2, be2), (W3, g3, be3)):
        hw = _matmul(h, W)
        a = _agg(hw, src, dst, dis)
        st = _stats(a)
        h = _bngelu(a, st, g, be)

    return _pool_head(h, batch, Wout, bout)


# speedup vs baseline: 3.9294x; 2.5566x over previous
import functools

import jax
import jax.numpy as jnp
from jax import lax
from jax.experimental import pallas as pl
from jax.experimental.pallas import tpu as pltpu
from jax.experimental.pallas import tpu_sc as plsc

N = 10000
G = 64
D = 512
NQ = 4
QD = D // NQ
ROWS = 1000
E = 160000
K = 128
NGROUPS = 1280
EPAD = K * NGROUPS
GPT = NGROUPS // 16
RPT = 624
RTAIL = N - 16 * RPT


def _mm_kernel(x_ref, w_ref, dis_ref, o_ref):
    h = jax.lax.dot_general(
        x_ref[...], w_ref[...], (((1,), (1,)), ((), ())),
        preferred_element_type=jnp.float32,
        precision=jax.lax.Precision.DEFAULT,
    ) * dis_ref[...]
    for q in range(NQ):
        o_ref[q] = h[:, q * QD:(q + 1) * QD]


def _matmul_q(x, W, dis):
    n, d_in = x.shape
    return pl.pallas_call(
        _mm_kernel,
        grid=(n // ROWS,),
        in_specs=[
            pl.BlockSpec((ROWS, d_in), lambda i: (i, 0)),
            pl.BlockSpec((D, d_in), lambda i: (0, 0)),
            pl.BlockSpec((ROWS, 1), lambda i: (i, 0)),
        ],
        out_specs=pl.BlockSpec((NQ, ROWS, QD), lambda i: (0, i, 0)),
        out_shape=jax.ShapeDtypeStruct((NQ, n, QD), jnp.float32),
    )(x, W, dis)


def _sc_agg_body(hs_ref, src_ref, dst_ref, out_ref, accum, sidx, didx, rows, sem):
    c = lax.axis_index("c")
    s = lax.axis_index("s")
    for p in range(2):
        q = c * 2 + p
        row0 = q * N + s * RPT
        pltpu.sync_copy(hs_ref.at[pl.ds(row0, RPT)], accum.at[pl.ds(s * RPT, RPT)])

        @pl.when(s == 15)
        def _():
            pltpu.sync_copy(hs_ref.at[pl.ds(q * N + 16 * RPT, RTAIL)],
                            accum.at[pl.ds(16 * RPT, RTAIL)])

        plsc.subcore_barrier()

        off = q * N

        @pl.loop(0, GPT)
        def _(m):
            g = (s * GPT + m) * K
            pltpu.sync_copy(src_ref.at[pl.ds(g, K)], sidx)
            pltpu.sync_copy(dst_ref.at[pl.ds(g, K)], didx)
            for j in range(K // 16):
                sl = pl.ds(j * 16, 16)
                sidx[sl] = sidx[sl] + off
            pltpu.async_copy(hs_ref.at[sidx], rows, sem).wait()
            pltpu.sync_copy(rows, accum.at[didx], add=True)

        plsc.subcore_barrier()
        pltpu.sync_copy(accum.at[pl.ds(s * RPT, RPT)], out_ref.at[pl.ds(row0, RPT)])

        @pl.when(s == 15)
        def _():
            pltpu.sync_copy(accum.at[pl.ds(16 * RPT, RTAIL)],
                            out_ref.at[pl.ds(q * N + 16 * RPT, RTAIL)])

        plsc.subcore_barrier()


def _sc_agg(hs, srcs, dsts):
    mesh = plsc.VectorSubcoreMesh(core_axis_name="c", subcore_axis_name="s")
    f = pl.kernel(
        _sc_agg_body,
        mesh=mesh,
        out_type=jax.ShapeDtypeStruct((NQ * N, QD), jnp.float32),
        scratch_types=[
            pltpu.VMEM_SHARED((N + 8, QD), jnp.float32),
            pltpu.VMEM((K,), jnp.int32),
            pltpu.VMEM((K,), jnp.int32),
            pltpu.VMEM((K, QD), jnp.float32),
            pltpu.SemaphoreType.DMA,
        ],
    )
    return f(hs, srcs, dsts)


def _stats_kernel(a_ref, dis_ref, o_ref):
    i = pl.program_id(0)

    @pl.when(i == 0)
    def _():
        o_ref[...] = jnp.zeros_like(o_ref)

    a = jnp.concatenate([a_ref[q] for q in range(NQ)], axis=1) * dis_ref[...]
    o_ref[0:1, :] += jnp.sum(a, axis=0, keepdims=True)
    o_ref[1:2, :] += jnp.sum(a * a, axis=0, keepdims=True)


def _stats_q(aq, dis):
    n = aq.shape[1]
    return pl.pallas_call(
        _stats_kernel,
        grid=(n // ROWS,),
        in_specs=[
            pl.BlockSpec((NQ, ROWS, QD), lambda i: (0, i, 0)),
            pl.BlockSpec((ROWS, 1), lambda i: (i, 0)),
        ],
        out_specs=pl.BlockSpec((8, D), lambda i: (0, 0)),
        out_shape=jax.ShapeDtypeStruct((8, D), jnp.float32),
    )(aq, dis)


def _bngelu_kernel(a_ref, dis_ref, st_ref, g_ref, be_ref, o_ref, *, n):
    mean = st_ref[0:1, :] / n
    var = st_ref[1:2, :] / n - mean * mean
    inv = jax.lax.rsqrt(var + 1e-5)
    a = jnp.concatenate([a_ref[q] for q in range(NQ)], axis=1) * dis_ref[...]
    y = (a - mean) * (inv * g_ref[...]) + be_ref[...]
    o_ref[...] = jax.nn.gelu(y)


def _bngelu_q(aq, dis, st, g, be):
    n = aq.shape[1]
    return pl.pallas_call(
        functools.partial(_bngelu_kernel, n=n),
        grid=(n // ROWS,),
        in_specs=[
            pl.BlockSpec((NQ, ROWS, QD), lambda i: (0, i, 0)),
            pl.BlockSpec((ROWS, 1), lambda i: (i, 0)),
            pl.BlockSpec((8, D), lambda i: (0, 0)),
            pl.BlockSpec((1, D), lambda i: (0, 0)),
            pl.BlockSpec((1, D), lambda i: (0, 0)),
        ],
        out_specs=pl.BlockSpec((ROWS, D), lambda i: (i, 0)),
        out_shape=jax.ShapeDtypeStruct((n, D), jnp.float32),
    )(aq, dis, st, g.reshape(1, D), be.reshape(1, D))


def _pool_kernel(h_ref, w_ref, b_ref, bout_ref, o_ref):
    i = pl.program_id(0)

    @pl.when(i == 0)
    def _():
        o_ref[...] = jnp.broadcast_to(bout_ref[...], o_ref.shape)

    s = jax.lax.dot_general(
        h_ref[...], w_ref[...], (((1,), (1,)), ((), ())),
        preferred_element_type=jnp.float32,
        precision=jax.lax.Precision.DEFAULT,
    )
    seg = jax.lax.broadcasted_iota(jnp.int32, (h_ref.shape[0], G), 1)
    onehot = (b_ref[0, 0, :].reshape(-1, 1) == seg).astype(jnp.float32)
    o_ref[...] += jax.lax.dot_general(
        s, onehot, (((0,), (0,)), ((), ())),
        preferred_element_type=jnp.float32,
        precision=jax.lax.Precision.DEFAULT,
    )


def _pool_head(h, batch, Wout, bout):
    n, d = h.shape
    out = pl.pallas_call(
        _pool_kernel,
        grid=(n // ROWS,),
        in_specs=[
            pl.BlockSpec((ROWS, d), lambda i: (i, 0)),
            pl.BlockSpec((1, d), lambda i: (0, 0)),
            pl.BlockSpec((1, 1, ROWS), lambda i: (i, 0, 0)),
            pl.BlockSpec((1, 1), lambda i: (0, 0)),
        ],
        out_specs=pl.BlockSpec((1, G), lambda i: (0, 0)),
        out_shape=jax.ShapeDtypeStruct((1, G), jnp.float32),
    )(h, Wout, batch.reshape(n // ROWS, 1, ROWS), bout.reshape(1, 1))
    return out.reshape(G)


def kernel(x, edge_index, batch, W0, b0, g0, be0, W1, b1, g1, be1,
           W2, b2, g2, be2, W3, b3, g3, be3, Wout, bout):
    src = edge_index[0]
    dst = edge_index[1]
    deg = jnp.ones((N,), jnp.float32).at[dst].add(1.0)
    dis = jax.lax.rsqrt(deg).reshape(N, 1)

    pad = EPAD - E
    srcs = jnp.concatenate([src, jnp.zeros((pad,), src.dtype)])
    dsts = jnp.concatenate([dst, jnp.full((pad,), N, dst.dtype)])

    h = x
    for W, g, be in ((W0, g0, be0), (W1, g1, be1), (W2, g2, be2), (W3, g3, be3)):
        hsq = _matmul_q(h, W, dis)
        aggq = _sc_agg(hsq.reshape(NQ * N, QD), srcs, dsts).reshape(NQ, N, QD)
        st = _stats_q(aggq, dis)
        h = _bngelu_q(aggq, dis, st, g, be)

    return _pool_head(h, batch, Wout, bout)


# speedup vs baseline: 4.7102x; 1.1987x over previous
import functools

import jax
import jax.numpy as jnp
from jax import lax
from jax.experimental import pallas as pl
from jax.experimental.pallas import tpu as pltpu
from jax.experimental.pallas import tpu_sc as plsc

N = 10000
G = 64
D = 512
NQ = 4
QD = D // NQ
ROWS = 1000
E = 160000
K = 64
NGROUPS = 2560
EPAD = K * NGROUPS
GPT = NGROUPS // 16
RPT = 624
RTAIL = N - 16 * RPT


def _mm_kernel(x_ref, w_ref, dis_ref, o_ref):
    h = jax.lax.dot_general(
        x_ref[...], w_ref[...], (((1,), (1,)), ((), ())),
        preferred_element_type=jnp.float32,
        precision=jax.lax.Precision.DEFAULT,
    ) * dis_ref[...]
    for q in range(NQ):
        o_ref[q] = h[:, q * QD:(q + 1) * QD]


def _matmul_q(x, W, dis):
    n, d_in = x.shape
    return pl.pallas_call(
        _mm_kernel,
        grid=(n // ROWS,),
        in_specs=[
            pl.BlockSpec((ROWS, d_in), lambda i: (i, 0)),
            pl.BlockSpec((D, d_in), lambda i: (0, 0)),
            pl.BlockSpec((ROWS, 1), lambda i: (i, 0)),
        ],
        out_specs=pl.BlockSpec((NQ, ROWS, QD), lambda i: (0, i, 0)),
        out_shape=jax.ShapeDtypeStruct((NQ, n, QD), jnp.float32),
    )(x, W, dis)


NI = 8
NR = 4
UNROLL = 8


def _sc_agg_body(hs_ref, src4_ref, dst_ref, out_ref, accum, sidx, didx, rows,
                 isem, gsem, ssem):
    c = lax.axis_index("c")
    s = lax.axis_index("s")

    for p in range(2):
        q = c * 2 + p
        row0 = q * N + s * RPT
        pltpu.sync_copy(hs_ref.at[pl.ds(row0, RPT)], accum.at[pl.ds(s * RPT, RPT)])

        @pl.when(s == 15)
        def _():
            pltpu.sync_copy(hs_ref.at[pl.ds(q * N + 16 * RPT, RTAIL)],
                            accum.at[pl.ds(16 * RPT, RTAIL)])

        plsc.subcore_barrier()

        ebase = q * EPAD
        g0 = s * GPT

        def idx_start(mm, bi):
            off = (g0 + mm) * K
            pltpu.async_copy(src4_ref.at[pl.ds(ebase + off, K)], sidx.at[bi],
                             isem.at[bi])
            pltpu.async_copy(dst_ref.at[pl.ds(off, K)], didx.at[bi], isem.at[bi])

        def idx_wait(mm, bi):
            off = (g0 + mm) * K
            pltpu.make_async_copy(src4_ref.at[pl.ds(ebase + off, K)],
                                  sidx.at[bi], isem.at[bi]).wait()
            pltpu.make_async_copy(dst_ref.at[pl.ds(off, K)], didx.at[bi],
                                  isem.at[bi]).wait()

        def gather_start(bi, br):
            pltpu.async_copy(hs_ref.at[sidx.at[bi]], rows.at[br], gsem.at[br])

        def gather_wait(bi, br):
            pltpu.make_async_copy(hs_ref.at[sidx.at[bi]], rows.at[br],
                                  gsem.at[br]).wait()

        def scat_start(bi, br):
            pltpu.async_copy(rows.at[br], accum.at[didx.at[bi]], ssem.at[br],
                             add=True)

        def scat_wait(bi, br):
            pltpu.make_async_copy(rows.at[br], accum.at[didx.at[bi]],
                                  ssem.at[br]).wait()

        for mm in range(3):
            idx_start(mm, mm % NI)
        idx_wait(0, 0)
        gather_start(0, 0)

        @pl.loop(0, GPT, step=UNROLL)
        def _(m0):
            for u in range(UNROLL):
                mm = m0 + u
                bi, br = u % NI, u % NR
                gather_wait(bi, br)
                scat_start(bi, br)
                bi1, br1 = (u + 1) % NI, (u + 1) % NR

                @pl.when(mm >= NR - 1)
                def _():
                    scat_wait((u + 1 - NR) % NI, br1)

                @pl.when(mm + 1 < GPT)
                def _():
                    idx_wait(mm + 1, bi1)
                    gather_start(bi1, br1)

                @pl.when(mm + 3 < GPT)
                def _():
                    idx_start(mm + 3, (u + 3) % NI)

        for mm in range(GPT - NR + 1, GPT):
            scat_wait(mm % NI, mm % NR)

        plsc.subcore_barrier()
        pltpu.sync_copy(accum.at[pl.ds(s * RPT, RPT)], out_ref.at[pl.ds(row0, RPT)])

        @pl.when(s == 15)
        def _():
            pltpu.sync_copy(accum.at[pl.ds(16 * RPT, RTAIL)],
                            out_ref.at[pl.ds(q * N + 16 * RPT, RTAIL)])

        plsc.subcore_barrier()


def _sc_agg(hs, src4, dsts):
    mesh = plsc.VectorSubcoreMesh(core_axis_name="c", subcore_axis_name="s")
    f = pl.kernel(
        _sc_agg_body,
        mesh=mesh,
        out_type=jax.ShapeDtypeStruct((NQ * N, QD), jnp.float32),
        scratch_types=[
            pltpu.VMEM_SHARED((N + 8, QD), jnp.float32),
            pltpu.VMEM((NI, K), jnp.int32),
            pltpu.VMEM((NI, K), jnp.int32),
            pltpu.VMEM((NR, K, QD), jnp.float32),
            pltpu.SemaphoreType.DMA((NI,)),
            pltpu.SemaphoreType.DMA((NR,)),
            pltpu.SemaphoreType.DMA((NR,)),
        ],
    )
    return f(hs, src4, dsts)


def _stats_kernel(a_ref, dis_ref, o_ref):
    i = pl.program_id(0)

    @pl.when(i == 0)
    def _():
        o_ref[...] = jnp.zeros_like(o_ref)

    a = jnp.concatenate([a_ref[q] for q in range(NQ)], axis=1) * dis_ref[...]
    o_ref[0:1, :] += jnp.sum(a, axis=0, keepdims=True)
    o_ref[1:2, :] += jnp.sum(a * a, axis=0, keepdims=True)


def _stats_q(aq, dis):
    n = aq.shape[1]
    return pl.pallas_call(
        _stats_kernel,
        grid=(n // ROWS,),
        in_specs=[
            pl.BlockSpec((NQ, ROWS, QD), lambda i: (0, i, 0)),
            pl.BlockSpec((ROWS, 1), lambda i: (i, 0)),
        ],
        out_specs=pl.BlockSpec((8, D), lambda i: (0, 0)),
        out_shape=jax.ShapeDtypeStruct((8, D), jnp.float32),
    )(aq, dis)


def _bngelu_kernel(a_ref, dis_ref, st_ref, g_ref, be_ref, o_ref, *, n):
    mean = st_ref[0:1, :] / n
    var = st_ref[1:2, :] / n - mean * mean
    inv = jax.lax.rsqrt(var + 1e-5)
    a = jnp.concatenate([a_ref[q] for q in range(NQ)], axis=1) * dis_ref[...]
    y = (a - mean) * (inv * g_ref[...]) + be_ref[...]
    o_ref[...] = jax.nn.gelu(y)


def _bngelu_q(aq, dis, st, g, be):
    n = aq.shape[1]
    return pl.pallas_call(
        functools.partial(_bngelu_kernel, n=n),
        grid=(n // ROWS,),
        in_specs=[
            pl.BlockSpec((NQ, ROWS, QD), lambda i: (0, i, 0)),
            pl.BlockSpec((ROWS, 1), lambda i: (i, 0)),
            pl.BlockSpec((8, D), lambda i: (0, 0)),
            pl.BlockSpec((1, D), lambda i: (0, 0)),
            pl.BlockSpec((1, D), lambda i: (0, 0)),
        ],
        out_specs=pl.BlockSpec((ROWS, D), lambda i: (i, 0)),
        out_shape=jax.ShapeDtypeStruct((n, D), jnp.float32),
    )(aq, dis, st, g.reshape(1, D), be.reshape(1, D))


def _pool_kernel(h_ref, w_ref, b_ref, bout_ref, o_ref):
    i = pl.program_id(0)

    @pl.when(i == 0)
    def _():
        o_ref[...] = jnp.broadcast_to(bout_ref[...], o_ref.shape)

    s = jax.lax.dot_general(
        h_ref[...], w_ref[...], (((1,), (1,)), ((), ())),
        preferred_element_type=jnp.float32,
        precision=jax.lax.Precision.DEFAULT,
    )
    seg = jax.lax.broadcasted_iota(jnp.int32, (h_ref.shape[0], G), 1)
    onehot = (b_ref[0, 0, :].reshape(-1, 1) == seg).astype(jnp.float32)
    o_ref[...] += jax.lax.dot_general(
        s, onehot, (((0,), (0,)), ((), ())),
        preferred_element_type=jnp.float32,
        precision=jax.lax.Precision.DEFAULT,
    )


def _pool_head(h, batch, Wout, bout):
    n, d = h.shape
    out = pl.pallas_call(
        _pool_kernel,
        grid=(n // ROWS,),
        in_specs=[
            pl.BlockSpec((ROWS, d), lambda i: (i, 0)),
            pl.BlockSpec((1, d), lambda i: (0, 0)),
            pl.BlockSpec((1, 1, ROWS), lambda i: (i, 0, 0)),
            pl.BlockSpec((1, 1), lambda i: (0, 0)),
        ],
        out_specs=pl.BlockSpec((1, G), lambda i: (0, 0)),
        out_shape=jax.ShapeDtypeStruct((1, G), jnp.float32),
    )(h, Wout, batch.reshape(n // ROWS, 1, ROWS), bout.reshape(1, 1))
    return out.reshape(G)


def kernel(x, edge_index, batch, W0, b0, g0, be0, W1, b1, g1, be1,
           W2, b2, g2, be2, W3, b3, g3, be3, Wout, bout):
    src = edge_index[0]
    dst = edge_index[1]
    deg = jnp.ones((N,), jnp.float32).at[dst].add(1.0)
    dis = jax.lax.rsqrt(deg).reshape(N, 1)

    pad = EPAD - E
    srcs = jnp.concatenate([src, jnp.zeros((pad,), src.dtype)])
    dsts = jnp.concatenate([dst, jnp.full((pad,), N, dst.dtype)])
    src4 = (srcs[None, :].astype(jnp.int32)
            + (jnp.arange(NQ, dtype=jnp.int32) * N)[:, None]).reshape(NQ * EPAD)

    h = x
    for W, g, be in ((W0, g0, be0), (W1, g1, be1), (W2, g2, be2), (W3, g3, be3)):
        hsq = _matmul_q(h, W, dis)
        aggq = _sc_agg(hsq.reshape(NQ * N, QD), src4, dsts).reshape(NQ, N, QD)
        st = _stats_q(aggq, dis)
        h = _bngelu_q(aggq, dis, st, g, be)

    return _pool_head(h, batch, Wout, bout)


# speedup vs baseline: 5.6139x; 1.1919x over previous
import functools

import jax
import jax.numpy as jnp
from jax import lax
from jax.experimental import pallas as pl
from jax.experimental.pallas import tpu as pltpu
from jax.experimental.pallas import tpu_sc as plsc

N = 10000
G = 64
D = 512
NQ = 4
QD = D // NQ
ROWS = 1000
E = 160000
K = 64
NGROUPS = 2560
EPAD = K * NGROUPS
GPT = NGROUPS // 16
RPT = 624
RTAIL = N - 16 * RPT


def _mm_kernel(x_ref, w_ref, dis_ref, o_ref):
    h = jax.lax.dot_general(
        x_ref[...], w_ref[...], (((1,), (1,)), ((), ())),
        preferred_element_type=jnp.float32,
        precision=jax.lax.Precision.DEFAULT,
    ) * dis_ref[...]
    for q in range(NQ):
        o_ref[q] = h[:, q * QD:(q + 1) * QD]


def _matmul_q(x, W, dis):
    n, d_in = x.shape
    return pl.pallas_call(
        _mm_kernel,
        grid=(n // ROWS,),
        in_specs=[
            pl.BlockSpec((ROWS, d_in), lambda i: (i, 0)),
            pl.BlockSpec((D, d_in), lambda i: (0, 0)),
            pl.BlockSpec((ROWS, 1), lambda i: (i, 0)),
        ],
        out_specs=pl.BlockSpec((NQ, ROWS, QD), lambda i: (0, i, 0)),
        out_shape=jax.ShapeDtypeStruct((NQ, n, QD), jnp.float32),
    )(x, W, dis)


NI = 8
NR = 4
UNROLL = 8


def _sc_agg_body(hs_ref, src4_ref, dst_ref, out_ref, accum, sidx, didx, rows,
                 isem, gsem, ssem):
    c = lax.axis_index("c")
    s = lax.axis_index("s")

    for p in range(2):
        q = c * 2 + p
        row0 = q * N + s * RPT
        pltpu.sync_copy(hs_ref.at[pl.ds(row0, RPT)], accum.at[pl.ds(s * RPT, RPT)])

        @pl.when(s == 15)
        def _():
            pltpu.sync_copy(hs_ref.at[pl.ds(q * N + 16 * RPT, RTAIL)],
                            accum.at[pl.ds(16 * RPT, RTAIL)])

        plsc.subcore_barrier()

        ebase = q * EPAD
        g0 = s * GPT

        def idx_start(mm, bi):
            off = (g0 + mm) * K
            pltpu.async_copy(src4_ref.at[pl.ds(ebase + off, K)], sidx.at[bi],
                             isem.at[bi])
            pltpu.async_copy(dst_ref.at[pl.ds(off, K)], didx.at[bi], isem.at[bi])

        def idx_wait(mm, bi):
            off = (g0 + mm) * K
            pltpu.make_async_copy(src4_ref.at[pl.ds(ebase + off, K)],
                                  sidx.at[bi], isem.at[bi]).wait()
            pltpu.make_async_copy(dst_ref.at[pl.ds(off, K)], didx.at[bi],
                                  isem.at[bi]).wait()

        def gather_start(bi, br):
            pltpu.async_copy(hs_ref.at[sidx.at[bi]], rows.at[br], gsem.at[br])

        def gather_wait(bi, br):
            pltpu.make_async_copy(hs_ref.at[sidx.at[bi]], rows.at[br],
                                  gsem.at[br]).wait()

        def scat_start(bi, br):
            pltpu.async_copy(rows.at[br], accum.at[didx.at[bi]], ssem.at[br],
                             add=True)

        def scat_wait(bi, br):
            pltpu.make_async_copy(rows.at[br], accum.at[didx.at[bi]],
                                  ssem.at[br]).wait()

        for mm in range(5):
            idx_start(mm, mm % NI)
        for mm in range(3):
            idx_wait(mm, mm % NI)
            gather_start(mm % NI, mm % NR)

        @pl.loop(0, GPT, step=UNROLL)
        def _(m0):
            for u in range(UNROLL):
                mm = m0 + u
                bi, br = u % NI, u % NR
                gather_wait(bi, br)
                scat_start(bi, br)

                @pl.when(mm >= 1)
                def _():
                    scat_wait((u - 1) % NI, (u - 1) % NR)

                @pl.when(mm + 3 < GPT)
                def _():
                    idx_wait(mm + 3, (u + 3) % NI)
                    gather_start((u + 3) % NI, (u + 3) % NR)

                @pl.when(mm + 5 < GPT)
                def _():
                    idx_start(mm + 5, (u + 5) % NI)

        scat_wait((GPT - 1) % NI, (GPT - 1) % NR)
        plsc.subcore_barrier()
        pltpu.sync_copy(accum.at[pl.ds(s * RPT, RPT)], out_ref.at[pl.ds(row0, RPT)])

        @pl.when(s == 15)
        def _():
            pltpu.sync_copy(accum.at[pl.ds(16 * RPT, RTAIL)],
                            out_ref.at[pl.ds(q * N + 16 * RPT, RTAIL)])

        plsc.subcore_barrier()


def _sc_agg(hs, src4, dsts):
    mesh = plsc.VectorSubcoreMesh(core_axis_name="c", subcore_axis_name="s")
    f = pl.kernel(
        _sc_agg_body,
        mesh=mesh,
        out_type=jax.ShapeDtypeStruct((NQ * N, QD), jnp.float32),
        scratch_types=[
            pltpu.VMEM_SHARED((N + 8, QD), jnp.float32),
            pltpu.VMEM((NI, K), jnp.int32),
            pltpu.VMEM((NI, K), jnp.int32),
            pltpu.VMEM((NR, K, QD), jnp.float32),
            pltpu.SemaphoreType.DMA((NI,)),
            pltpu.SemaphoreType.DMA((NR,)),
            pltpu.SemaphoreType.DMA((NR,)),
        ],
    )
    return f(hs, src4, dsts)


def _stats_kernel(a_ref, dis_ref, o_ref):
    i = pl.program_id(0)

    @pl.when(i == 0)
    def _():
        o_ref[...] = jnp.zeros_like(o_ref)

    a = jnp.concatenate([a_ref[q] for q in range(NQ)], axis=1) * dis_ref[...]
    o_ref[0:1, :] += jnp.sum(a, axis=0, keepdims=True)
    o_ref[1:2, :] += jnp.sum(a * a, axis=0, keepdims=True)


def _stats_q(aq, dis):
    n = aq.shape[1]
    return pl.pallas_call(
        _stats_kernel,
        grid=(n // ROWS,),
        in_specs=[
            pl.BlockSpec((NQ, ROWS, QD), lambda i: (0, i, 0)),
            pl.BlockSpec((ROWS, 1), lambda i: (i, 0)),
        ],
        out_specs=pl.BlockSpec((8, D), lambda i: (0, 0)),
        out_shape=jax.ShapeDtypeStruct((8, D), jnp.float32),
    )(aq, dis)


def _bngelu_kernel(a_ref, dis_ref, st_ref, g_ref, be_ref, o_ref, *, n):
    mean = st_ref[0:1, :] / n
    var = st_ref[1:2, :] / n - mean * mean
    inv = jax.lax.rsqrt(var + 1e-5)
    a = jnp.concatenate([a_ref[q] for q in range(NQ)], axis=1) * dis_ref[...]
    y = (a - mean) * (inv * g_ref[...]) + be_ref[...]
    o_ref[...] = jax.nn.gelu(y)


def _bngelu_q(aq, dis, st, g, be):
    n = aq.shape[1]
    return pl.pallas_call(
        functools.partial(_bngelu_kernel, n=n),
        grid=(n // ROWS,),
        in_specs=[
            pl.BlockSpec((NQ, ROWS, QD), lambda i: (0, i, 0)),
            pl.BlockSpec((ROWS, 1), lambda i: (i, 0)),
            pl.BlockSpec((8, D), lambda i: (0, 0)),
            pl.BlockSpec((1, D), lambda i: (0, 0)),
            pl.BlockSpec((1, D), lambda i: (0, 0)),
        ],
        out_specs=pl.BlockSpec((ROWS, D), lambda i: (i, 0)),
        out_shape=jax.ShapeDtypeStruct((n, D), jnp.float32),
    )(aq, dis, st, g.reshape(1, D), be.reshape(1, D))


def _pool_kernel(h_ref, w_ref, b_ref, bout_ref, o_ref):
    i = pl.program_id(0)

    @pl.when(i == 0)
    def _():
        o_ref[...] = jnp.broadcast_to(bout_ref[...], o_ref.shape)

    s = jax.lax.dot_general(
        h_ref[...], w_ref[...], (((1,), (1,)), ((), ())),
        preferred_element_type=jnp.float32,
        precision=jax.lax.Precision.DEFAULT,
    )
    seg = jax.lax.broadcasted_iota(jnp.int32, (h_ref.shape[0], G), 1)
    onehot = (b_ref[0, 0, :].reshape(-1, 1) == seg).astype(jnp.float32)
    o_ref[...] += jax.lax.dot_general(
        s, onehot, (((0,), (0,)), ((), ())),
        preferred_element_type=jnp.float32,
        precision=jax.lax.Precision.DEFAULT,
    )


def _pool_head(h, batch, Wout, bout):
    n, d = h.shape
    out = pl.pallas_call(
        _pool_kernel,
        grid=(n // ROWS,),
        in_specs=[
            pl.BlockSpec((ROWS, d), lambda i: (i, 0)),
            pl.BlockSpec((1, d), lambda i: (0, 0)),
            pl.BlockSpec((1, 1, ROWS), lambda i: (i, 0, 0)),
            pl.BlockSpec((1, 1), lambda i: (0, 0)),
        ],
        out_specs=pl.BlockSpec((1, G), lambda i: (0, 0)),
        out_shape=jax.ShapeDtypeStruct((1, G), jnp.float32),
    )(h, Wout, batch.reshape(n // ROWS, 1, ROWS), bout.reshape(1, 1))
    return out.reshape(G)


def kernel(x, edge_index, batch, W0, b0, g0, be0, W1, b1, g1, be1,
           W2, b2, g2, be2, W3, b3, g3, be3, Wout, bout):
    src = edge_index[0]
    dst = edge_index[1]
    deg = jnp.ones((N,), jnp.float32).at[dst].add(1.0)
    dis = jax.lax.rsqrt(deg).reshape(N, 1)

    pad = EPAD - E
    srcs = jnp.concatenate([src, jnp.zeros((pad,), src.dtype)])
    dsts = jnp.concatenate([dst, jnp.full((pad,), N, dst.dtype)])
    src4 = (srcs[None, :].astype(jnp.int32)
            + (jnp.arange(NQ, dtype=jnp.int32) * N)[:, None]).reshape(NQ * EPAD)

    h = x
    for W, g, be in ((W0, g0, be0), (W1, g1, be1), (W2, g2, be2), (W3, g3, be3)):
        hsq = _matmul_q(h, W, dis)
        aggq = _sc_agg(hsq.reshape(NQ * N, QD), src4, dsts).reshape(NQ, N, QD)
        st = _stats_q(aggq, dis)
        h = _bngelu_q(aggq, dis, st, g, be)

    return _pool_head(h, batch, Wout, bout)


# speedup vs baseline: 7.9155x; 1.4100x over previous
import dataclasses
import functools

import jax
import jax.numpy as jnp
from jax import lax
from jax.experimental import pallas as pl
from jax.experimental.pallas import tpu as pltpu
from jax.experimental.pallas import tpu_sc as plsc

N = 10000
NPAD = 10240
RNG = NPAD // 4
G = 64
D = 512
SL = 4
ROWS = 1000
MROWS = 1024
E = 160000
K = 32
RPT = RNG // 16
NI = 8
NR = 2
UNROLL = 8


def _mm_kernel(x_ref, w_ref, dis_ref, o_ref):
    o_ref[...] = jax.lax.dot_general(
        x_ref[...], w_ref[...], (((1,), (1,)), ((), ())),
        preferred_element_type=jnp.float32,
        precision=jax.lax.Precision.DEFAULT,
    ) * dis_ref[...]


def _matmul(x, W, dis):
    n, d_in = x.shape
    return pl.pallas_call(
        _mm_kernel,
        grid=(n // MROWS,),
        in_specs=[
            pl.BlockSpec((MROWS, d_in), lambda i: (i, 0)),
            pl.BlockSpec((D, d_in), lambda i: (0, 0)),
            pl.BlockSpec((MROWS, 1), lambda i: (i, 0)),
        ],
        out_specs=pl.BlockSpec((MROWS, D), lambda i: (i, 0)),
        out_shape=jax.ShapeDtypeStruct((n, D), jnp.float32),
    )(x, W, dis)


def _sc_agg_body(hs_ref, src_ref, dst_ref, gb_ref, out_ref,
                 accum, gb, sidx, didx, rows, isem, gsem, ssem):
    c = lax.axis_index("c")
    s = lax.axis_index("s")
    pltpu.sync_copy(gb_ref.at[c], gb)

    for p in range(2):
        base = (c * 2 + p) * RNG
        pltpu.sync_copy(hs_ref.at[pl.ds(base + s * RPT, RPT)],
                        accum.at[pl.ds(s * RPT, RPT)])
        plsc.subcore_barrier()

        gbv = gb[...]
        glo = gbv[p]
        ghi = gbv[p + 2]
        ngt = (ghi - glo - s + 15) // 16
        ng8 = (ngt // UNROLL + 1) * UNROLL

        def goff(mm):
            return (glo + mm * 16 + s) * K

        def idx_start(mm, bi):
            off = goff(mm)
            pltpu.async_copy(src_ref.at[pl.ds(off, K)], sidx.at[bi], isem.at[bi])
            pltpu.async_copy(dst_ref.at[pl.ds(off, K)], didx.at[bi], isem.at[bi])

        def idx_wait(mm, bi):
            off = goff(mm)
            pltpu.make_async_copy(src_ref.at[pl.ds(off, K)], sidx.at[bi],
                                  isem.at[bi]).wait()
            pltpu.make_async_copy(dst_ref.at[pl.ds(off, K)], didx.at[bi],
                                  isem.at[bi]).wait()
            for j in range(K // 16):
                sl = pl.ds(j * 16, 16)
                dl = didx.at[bi][sl] - base
                ok = (dl >= 0) & (dl < RNG)
                didx.at[bi][sl] = jnp.where(ok, dl, RNG)

        def gather_start(bi, br):
            pltpu.async_copy(hs_ref.at[sidx.at[bi]], rows.at[br], gsem.at[br])

        def gather_wait(bi, br):
            pltpu.make_async_copy(hs_ref.at[sidx.at[bi]], rows.at[br],
                                  gsem.at[br]).wait()

        def scat_start(bi, br):
            pltpu.async_copy(rows.at[br], accum.at[didx.at[bi]], ssem.at[br],
                             add=True)

        def scat_wait(bi, br):
            pltpu.make_async_copy(rows.at[br], accum.at[didx.at[bi]],
                                  ssem.at[br]).wait()

        for j in range(3):
            @pl.when(j < ngt)
            def _():
                idx_start(j, j % NI)

        @pl.when(0 < ngt)
        def _():
            idx_wait(0, 0)
            gather_start(0, 0)

        @pl.loop(0, ng8, step=UNROLL)
        def _(m0):
            for u in range(UNROLL):
                mm = m0 + u
                bi, br = u % NI, u % NR

                @pl.when(mm < ngt)
                def _():
                    gather_wait(bi, br)
                    scat_start(bi, br)

                @pl.when((mm >= 1) & (mm <= ngt))
                def _():
                    scat_wait((u - 1) % NI, (u - 1) % NR)

                @pl.when(mm + 1 < ngt)
                def _():
                    idx_wait(mm + 1, (u + 1) % NI)
                    gather_start((u + 1) % NI, (u + 1) % NR)

                @pl.when(mm + 3 < ngt)
                def _():
                    idx_start(mm + 3, (u + 3) % NI)

        plsc.subcore_barrier()
        pltpu.sync_copy(accum.at[pl.ds(s * RPT, RPT)],
                        out_ref.at[pl.ds(base + s * RPT, RPT)])
        plsc.subcore_barrier()


def _sc_agg(hs, srcs, dsts, gbounds):
    mesh = plsc.VectorSubcoreMesh(core_axis_name="c", subcore_axis_name="s")
    f = pl.kernel(
        _sc_agg_body,
        mesh=mesh,
        out_type=jax.ShapeDtypeStruct((NPAD, SL, 128), jnp.float32),
        scratch_types=[
            pltpu.VMEM_SHARED((RNG + 8, SL, 128), jnp.float32),
            pltpu.VMEM((16,), jnp.int32),
            pltpu.VMEM((NI, K), jnp.int32),
            pltpu.VMEM((NI, K), jnp.int32),
            pltpu.VMEM((NR, K, SL, 128), jnp.float32),
            pltpu.SemaphoreType.DMA((NI,)),
            pltpu.SemaphoreType.DMA((NR,)),
            pltpu.SemaphoreType.DMA((NR,)),
        ],
    )
    return f(hs, srcs, dsts, gbounds)


def _stats_kernel(a_ref, dis_ref, o_ref):
    i = pl.program_id(0)

    @pl.when(i == 0)
    def _():
        o_ref[...] = jnp.zeros_like(o_ref)

    a = a_ref[...] * dis_ref[...]
    o_ref[0:1, :] += jnp.sum(a, axis=0, keepdims=True)
    o_ref[1:2, :] += jnp.sum(a * a, axis=0, keepdims=True)


def _stats(a, dis):
    return pl.pallas_call(
        _stats_kernel,
        grid=(N // ROWS,),
        in_specs=[
            pl.BlockSpec((ROWS, D), lambda i: (i, 0)),
            pl.BlockSpec((ROWS, 1), lambda i: (i, 0)),
        ],
        out_specs=pl.BlockSpec((8, D), lambda i: (0, 0)),
        out_shape=jax.ShapeDtypeStruct((8, D), jnp.float32),
    )(a, dis)


def _bngelu_kernel(a_ref, dis_ref, st_ref, g_ref, be_ref, o_ref):
    mean = st_ref[0:1, :] / N
    var = st_ref[1:2, :] / N - mean * mean
    inv = jax.lax.rsqrt(var + 1e-5)
    a = a_ref[...] * dis_ref[...]
    y = (a - mean) * (inv * g_ref[...]) + be_ref[...]
    o_ref[...] = jax.nn.gelu(y)


def _bngelu(a, dis, st, g, be):
    return pl.pallas_call(
        _bngelu_kernel,
        grid=(N // ROWS,),
        in_specs=[
            pl.BlockSpec((ROWS, D), lambda i: (i, 0)),
            pl.BlockSpec((ROWS, 1), lambda i: (i, 0)),
            pl.BlockSpec((8, D), lambda i: (0, 0)),
            pl.BlockSpec((1, D), lambda i: (0, 0)),
            pl.BlockSpec((1, D), lambda i: (0, 0)),
        ],
        out_specs=pl.BlockSpec((ROWS, D), lambda i: (i, 0)),
        out_shape=jax.ShapeDtypeStruct((NPAD, D), jnp.float32),
    )(a, dis, st, g.reshape(1, D), be.reshape(1, D))


def _pool_kernel(h_ref, w_ref, b_ref, bout_ref, o_ref):
    i = pl.program_id(0)

    @pl.when(i == 0)
    def _():
        o_ref[...] = jnp.broadcast_to(bout_ref[...], o_ref.shape)

    s = jax.lax.dot_general(
        h_ref[...], w_ref[...], (((1,), (1,)), ((), ())),
        preferred_element_type=jnp.float32,
        precision=jax.lax.Precision.DEFAULT,
    )
    seg = jax.lax.broadcasted_iota(jnp.int32, (h_ref.shape[0], G), 1)
    onehot = (b_ref[0, 0, :].reshape(-1, 1) == seg).astype(jnp.float32)
    o_ref[...] += jax.lax.dot_general(
        s, onehot, (((0,), (0,)), ((), ())),
        preferred_element_type=jnp.float32,
        precision=jax.lax.Precision.DEFAULT,
    )


def _pool_head(h, batch, Wout, bout):
    out = pl.pallas_call(
        _pool_kernel,
        grid=(N // ROWS,),
        in_specs=[
            pl.BlockSpec((ROWS, D), lambda i: (i, 0)),
            pl.BlockSpec((1, D), lambda i: (0, 0)),
            pl.BlockSpec((1, 1, ROWS), lambda i: (i, 0, 0)),
            pl.BlockSpec((1, 1), lambda i: (0, 0)),
        ],
        out_specs=pl.BlockSpec((1, G), lambda i: (0, 0)),
        out_shape=jax.ShapeDtypeStruct((1, G), jnp.float32),
    )(h, Wout, batch.reshape(N // ROWS, 1, ROWS), bout.reshape(1, 1))
    return out.reshape(G)


def kernel(x, edge_index, batch, W0, b0, g0, be0, W1, b1, g1, be1,
           W2, b2, g2, be2, W3, b3, g3, be3, Wout, bout):
    src = edge_index[0]
    dst = edge_index[1]
    deg = jnp.ones((N,), jnp.float32).at[dst].add(1.0)
    dis = jnp.pad(jax.lax.rsqrt(deg), (0, NPAD - N)).reshape(NPAD, 1)

    order = jnp.argsort(dst)
    srcs = src[order].astype(jnp.int32)
    dsts = dst[order].astype(jnp.int32)
    bnd = jnp.searchsorted(dsts, jnp.arange(5, dtype=jnp.int32) * RNG).astype(jnp.int32)
    glo = bnd[:4] // K
    ghi = -(-bnd[1:5] // K)
    z = jnp.zeros((12,), jnp.int32)
    gbounds = jnp.stack([
        jnp.concatenate([glo[0:2], ghi[0:2], z]),
        jnp.concatenate([glo[2:4], ghi[2:4], z])])

    h = jnp.pad(x, ((0, NPAD - N), (0, 0)))
    for W, g, be in ((W0, g0, be0), (W1, g1, be1), (W2, g2, be2), (W3, g3, be3)):
        hs = _matmul(h, W, dis)
        a = _sc_agg(hs.reshape(NPAD, SL, 128), srcs, dsts, gbounds)
        a = a.reshape(NPAD, D)
        st = _stats(a, dis)
        h = _bngelu(a, dis, st, g, be)

    return _pool_head(h, batch, Wout, bout)
